# baseline (speedup 1.0000x reference)
import jax
import jax.numpy as jnp
from jax.experimental import pallas as pl

_N = 16384
_R = 128
_C = 16
_NCHUNK = 128 // _C


def _listmle_body(tT_ref, t_ref, p_ref, out_ref):
    k = pl.program_id(0)
    p_all = p_ref[...]
    m = jnp.max(p_all)
    lane = jax.lax.broadcasted_iota(jnp.int32, (128, 128), 1)
    ti = jnp.sum(jnp.where(lane == k, tT_ref[...], 0.0),
                 axis=1, keepdims=True).reshape(_R, 1, 1)
    i_idx = k * _R + jax.lax.broadcasted_iota(jnp.int32, (_R, 1, 1), 0)

    def chunk(c, acc):
        tj = t_ref[pl.ds(c * _C, _C), :].reshape(1, _C, 128)
        pj = p_ref[pl.ds(c * _C, _C), :].reshape(1, _C, 128)
        ej = jnp.exp(pj - m)
        j_idx = (c * _C + jax.lax.broadcasted_iota(jnp.int32, (1, _C, 128), 1)) * 128 \
            + jax.lax.broadcasted_iota(jnp.int32, (1, _C, 128), 2)
        mask = (tj > ti) | ((tj == ti) & (j_idx >= i_idx))
        contrib = jnp.where(mask, ej, 0.0)
        return acc + jnp.sum(contrib, axis=(1, 2), keepdims=True)

    acc = jax.lax.fori_loop(0, _NCHUNK, chunk,
                            jnp.zeros((_R, 1, 1), jnp.float32))
    partial = jnp.sum(jnp.log(acc), axis=0)

    @pl.when(k == 0)
    def _():
        out_ref[...] = -(jnp.sum(p_all, axis=(0, 1), keepdims=True) - _N * m)

    out_ref[...] += partial

    @pl.when(k == pl.num_programs(0) - 1)
    def _():
        out_ref[...] = out_ref[...] / _N


def kernel(pred, target):
    t2 = target.reshape(128, 128)
    p2 = pred.reshape(128, 128)
    tT = t2.T
    out = pl.pallas_call(
        _listmle_body,
        grid=(_N // _R,),
        in_specs=[
            pl.BlockSpec((128, 128), lambda k: (0, 0)),
            pl.BlockSpec((128, 128), lambda k: (0, 0)),
            pl.BlockSpec((128, 128), lambda k: (0, 0)),
        ],
        out_specs=pl.BlockSpec((1, 1), lambda k: (0, 0)),
        out_shape=jax.ShapeDtypeStruct((1, 1), jnp.float32),
    )(tT, t2, p2)
    return out[0, 0]


# speedup vs baseline: 43.4904x; 43.4904x over previous
import jax
import jax.numpy as jnp
from jax.experimental import pallas as pl
from jax.experimental.pallas import tpu as pltpu

_N = 16384
_D = 128


def _partner(x, j, r_iota, c_iota):
    if j < _D:
        lo = pltpu.roll(x, _D - j, axis=1)
        hi = pltpu.roll(x, j, axis=1)
        take_lo = (c_iota & j) == 0
    else:
        d = j // _D
        lo = pltpu.roll(x, _D - d, axis=0)
        hi = pltpu.roll(x, d, axis=0)
        take_lo = (r_iota & d) == 0
    return jnp.where(take_lo, lo, hi)


def _bit_clear(k, r_iota, c_iota):
    if k < _D:
        return (c_iota & k) == 0
    return (r_iota & (k // _D)) == 0


def _listmle_body(t_ref, p_ref, out_ref):
    r_iota = jax.lax.broadcasted_iota(jnp.int32, (_D, _D), 0)
    c_iota = jax.lax.broadcasted_iota(jnp.int32, (_D, _D), 1)

    bits = jax.lax.bitcast_convert_type(t_ref[...], jnp.int32)
    key = jnp.where(bits < 0, bits ^ jnp.int32(0x7FFFFFFF), bits)
    val = p_ref[...]

    m = 2
    while m <= _N:
        j = m // 2
        while j >= 1:
            asc = _bit_clear(m, r_iota, c_iota) if m <= _N // 2 else None
            low = _bit_clear(j, r_iota, c_iota)
            want_min = low if asc is None else jnp.logical_not(
                jnp.logical_xor(asc, low))
            pk = _partner(key, j, r_iota, c_iota)
            pv = _partner(val, j, r_iota, c_iota)
            self_is_min = (key < pk) | ((key == pk) & low)
            take_self = jnp.logical_not(jnp.logical_xor(want_min, self_is_min))
            key = jnp.where(take_self, key, pk)
            val = jnp.where(take_self, val, pv)
            j //= 2
        m *= 2

    mx = jnp.max(val)
    shifted_sum = jnp.sum(val) - _N * mx
    e = jnp.exp(val - mx)

    a_iota = r_iota
    b_iota = c_iota
    upper = (a_iota >= b_iota).astype(jnp.float32)
    s_intra = jax.lax.dot_general(
        e, upper, (((1,), (0,)), ((), ())),
        preferred_element_type=jnp.float32,
        precision=jax.lax.Precision.HIGHEST)
    strict = (b_iota > a_iota).astype(jnp.float32)
    row_tot = jnp.sum(e, axis=1, keepdims=True)
    carry = jax.lax.dot_general(
        strict, row_tot, (((1,), (0,)), ((), ())),
        preferred_element_type=jnp.float32,
        precision=jax.lax.Precision.HIGHEST)
    s = s_intra + carry
    total = jnp.sum(jnp.log(s), axis=(0, 1), keepdims=True) - shifted_sum
    out_ref[...] = total.reshape(1, 1) / _N


def kernel(pred, target):
    t2 = target.reshape(_D, _D)
    p2 = pred.reshape(_D, _D)
    out = pl.pallas_call(
        _listmle_body,
        out_shape=jax.ShapeDtypeStruct((1, 1), jnp.float32),
    )(t2, p2)
    return out[0, 0]
